# baseline (speedup 1.0000x reference)
import functools

import jax
import jax.numpy as jnp
import numpy as np
from jax import lax
from jax.experimental import pallas as pl
from jax.experimental.pallas import tpu as pltpu
from jax.experimental.pallas import tpu_sc as plsc

N = 10000
E = 320000
H = 256
IN = 48
NG = 64
NPAD = 10240
EP = 327680
ER = EP // 128
BE = 2560

_F32 = jnp.float32


def _silu(v):
    return v * jax.nn.sigmoid(v)


def _base_body(d2_ref, wr_ref, wq_ref, wc_ref, bc_ref, out_ref):
    d2 = d2_ref[:]
    dist = jnp.sqrt(d2 + 1e-12)
    mu = (lax.broadcasted_iota(jnp.int32, (1, 9), 1).astype(_F32)
          * np.float32(0.75))
    sig = np.float32(6.0 / 9)
    rbf = jnp.exp(-(((dist - mu) / sig) ** 2))
    wr = wr_ref[0]
    wq = wq_ref[0]
    wc = wc_ref[0]
    bc = bc_ref[0]
    rbf_h = _silu(jnp.dot(rbf, wr, preferred_element_type=_F32))
    qr_h = _silu(dist * wq)
    bf = jnp.bfloat16
    base = (jnp.dot(rbf_h.astype(bf), wc[:H].astype(bf),
                    preferred_element_type=_F32)
            + jnp.dot(qr_h.astype(bf), wc[H:].astype(bf),
                      preferred_element_type=_F32) + bc)
    out_ref[0, 0] = base[:, :128]
    out_ref[0, 1] = base[:, 128:]


_base_call = pl.pallas_call(
    _base_body,
    grid=(3, EP // BE),
    in_specs=[
        pl.BlockSpec((BE, 1), lambda l, e: (e, 0)),
        pl.BlockSpec((1, 9, H), lambda l, e: (l, 0, 0)),
        pl.BlockSpec((1, 1, H), lambda l, e: (l, 0, 0)),
        pl.BlockSpec((1, 2 * H, H), lambda l, e: (l, 0, 0)),
        pl.BlockSpec((1, 1, H), lambda l, e: (l, 0, 0)),
    ],
    out_specs=pl.BlockSpec((1, 2, BE, 128), lambda l, e: (l, 0, e, 0)),
    out_shape=jax.ShapeDtypeStruct((3, 2, EP, 128), _F32),
)


_SC_MESH = plsc.VectorSubcoreMesh(core_axis_name="c", subcore_axis_name="s",
                                  num_cores=2, num_subcores=16)


def _edge_prep_body(src_ref, dst_ref, px_h, py_h, pz_h, d2_h, degp_h,
                    px, py, pz, srcv, dstv, d2v, onesv, zv, deg_sh):
    c = lax.axis_index("c")
    s = lax.axis_index("s")
    wid = s * 2 + c
    row0 = wid * 80
    pltpu.sync_copy(px_h, px)
    pltpu.sync_copy(py_h, py)
    pltpu.sync_copy(pz_h, pz)
    pltpu.sync_copy(src_ref.at[pl.ds(row0, 80), :], srcv)
    pltpu.sync_copy(dst_ref.at[pl.ds(row0, 80), :], dstv)

    def _zero(i, _):
        zv[pl.ds(i * 16, 16)] = jnp.zeros((16,), _F32)
        return _
    lax.fori_loop(0, 40, _zero, None)
    pltpu.sync_copy(zv, deg_sh.at[pl.ds(s * 640, 640)])

    def _one(i, _):
        onesv[pl.ds(i * 16, 16)] = jnp.full((16,), 1.0, _F32)
        return _
    lax.fori_loop(0, 8, _one, None)
    plsc.subcore_barrier()

    def _blk(k, _):
        def _grp(g, _):
            sl = pl.ds(g * 16, 16)
            si = srcv[k, sl]
            di = dstv[k, sl]
            dx = plsc.load_gather(px, [si]) - plsc.load_gather(px, [di])
            dy = plsc.load_gather(py, [si]) - plsc.load_gather(py, [di])
            dz = plsc.load_gather(pz, [si]) - plsc.load_gather(pz, [di])
            d2v[k, sl] = dx * dx + dy * dy + dz * dz
            return _
        lax.fori_loop(0, 8, _grp, None)
        pltpu.sync_copy(onesv, deg_sh.at[dstv.at[k]], add=True)
        return _
    lax.fori_loop(0, 80, _blk, None)
    pltpu.sync_copy(d2v, d2_h.at[pl.ds(row0, 80), :])
    plsc.subcore_barrier()

    @pl.when(s == 0)
    def _():
        pltpu.sync_copy(deg_sh, degp_h.at[c])


_edge_prep = pl.kernel(
    _edge_prep_body,
    out_type=(jax.ShapeDtypeStruct((ER, 128), _F32),
              jax.ShapeDtypeStruct((2, NPAD), _F32)),
    mesh=_SC_MESH,
    compiler_params=pltpu.CompilerParams(needs_layout_passes=False),
    scratch_types=[
        pltpu.VMEM((NPAD,), _F32),
        pltpu.VMEM((NPAD,), _F32),
        pltpu.VMEM((NPAD,), _F32),
        pltpu.VMEM((80, 128), jnp.int32),
        pltpu.VMEM((80, 128), jnp.int32),
        pltpu.VMEM((80, 128), _F32),
        pltpu.VMEM((128,), _F32),
        pltpu.VMEM((640,), _F32),
        pltpu.VMEM_SHARED((NPAD,), _F32),
    ],
)


_NBLK = 320


def _msg_agg_body(src_ref, dst_ref, h0_ref, h1_ref, base_ref, lrow_ref,
                  out_ref, ibs, ibd, rows, bbuf, sbuf, lbuf, bidx,
                  is0, is1, is2, is3, gs0, gs1, bs0, bs1, ss0, ss1, agg_sh):
    c = lax.axis_index("c")
    s = lax.axis_index("s")
    blk0 = s * _NBLK
    pltpu.sync_copy(lrow_ref, lbuf)
    iota16 = lax.broadcasted_iota(jnp.int32, (16,), 0)
    off_v = (lbuf[:] * 2 + c) * EP
    isems = (is0, is1, is2, is3)
    gsems = (gs0, gs1)
    bsems = (bs0, bs1)
    ssems = (ss0, ss1)

    def idx_issue(k, j):
        pltpu.async_copy(src_ref.at[blk0 + k], ibs.at[j], isems[j])
        pltpu.async_copy(dst_ref.at[blk0 + k], ibd.at[j], isems[j])

    def idx_wait(j):
        pltpu.make_async_copy(src_ref.at[blk0], ibs.at[j], isems[j]).wait()
        pltpu.make_async_copy(dst_ref.at[blk0], ibd.at[j], isems[j]).wait()

    def g_issue(b, j):
        @pl.when(c == 0)
        def _():
            pltpu.async_copy(h0_ref.at[ibs.at[j]], rows.at[b], gsems[b])

        @pl.when(c == 1)
        def _():
            pltpu.async_copy(h1_ref.at[ibs.at[j]], rows.at[b], gsems[b])

    def b_issue(k):
        e0 = (blk0 + k) * 64
        for g in range(4):
            bidx[0, pl.ds(g * 16, 16)] = off_v + (e0 + g * 16) + iota16
        pltpu.async_copy(base_ref.at[bidx.at[0]], bbuf, bs0)

    def gb_wait(b, j):
        pltpu.make_async_copy(h0_ref.at[ibs.at[j]], rows.at[b],
                              gsems[b]).wait()
        pltpu.make_async_copy(base_ref.at[bidx.at[0]], bbuf, bs0).wait()

    def scat_issue(b, j):
        pltpu.async_copy(sbuf.at[b], agg_sh.at[ibd.at[j]], ssems[b],
                         add=True)

    def scat_wait(b, j):
        pltpu.make_async_copy(sbuf.at[b], agg_sh.at[ibd.at[j]],
                              ssems[b]).wait()

    def process(k, r, first=False):
        b = r & 1
        if not first:
            scat_wait(b, (r + 2) & 3)
        if isinstance(k, int):
            if k + 2 < _NBLK:
                idx_issue(k + 2, (r + 2) & 3)
            if k + 1 < _NBLK:
                idx_wait((r + 1) & 3)
                g_issue(1 - b, (r + 1) & 3)
        else:
            @pl.when(k + 2 < _NBLK)
            def _():
                idx_issue(k + 2, (r + 2) & 3)

            @pl.when(k + 1 < _NBLK)
            def _():
                idx_wait((r + 1) & 3)
                g_issue(1 - b, (r + 1) & 3)
        gb_wait(b, r)
        @plsc.parallel_loop(0, 64, unroll=2)
        def _mrow(i):
            for j in range(8):
                sl = pl.ds(j * 16, 16)
                sbuf[b, i, sl] = rows[b, i, sl] * bbuf[i, sl]
        scat_issue(b, r)
        if isinstance(k, int):
            if k + 1 < _NBLK:
                b_issue(k + 1)
        else:
            @pl.when(k + 1 < _NBLK)
            def _():
                b_issue(k + 1)

    @plsc.parallel_loop(0, 64, unroll=2)
    def _zrow(i):
        for j in range(8):
            rows[0, i, pl.ds(j * 16, 16)] = jnp.zeros((16,), _F32)
    for j in range(10):
        pltpu.sync_copy(rows.at[0], agg_sh.at[pl.ds(s * 640 + j * 64, 64), :])
    plsc.subcore_barrier()

    idx_issue(0, 0)
    idx_issue(1, 1)
    idx_wait(0)
    g_issue(0, 0)
    b_issue(0)
    process(0, 0, first=True)
    process(1, 1, first=True)
    process(2, 2)
    process(3, 3)

    def _sb(it, _):
        k0 = it * 4
        for r in range(4):
            process(k0 + r, r)
        return _
    lax.fori_loop(1, _NBLK // 4, _sb, None)
    scat_wait(0, 2)
    scat_wait(1, 3)
    plsc.subcore_barrier()
    pltpu.sync_copy(agg_sh.at[pl.ds(s * 640, 640), :],
                    out_ref.at[c, pl.ds(s * 640, 640), :])


_msg_agg = pl.kernel(
    _msg_agg_body,
    out_type=jax.ShapeDtypeStruct((2, NPAD, 128), _F32),
    mesh=_SC_MESH,
    compiler_params=pltpu.CompilerParams(needs_layout_passes=False),
    scratch_types=[
        pltpu.VMEM((4, 64), jnp.int32),
        pltpu.VMEM((4, 64), jnp.int32),
        pltpu.VMEM((2, 64, 128), _F32),
        pltpu.VMEM((64, 128), _F32),
        pltpu.VMEM((2, 64, 128), _F32),
        pltpu.VMEM((16,), jnp.int32),
        pltpu.VMEM((1, 64), jnp.int32),
        pltpu.SemaphoreType.DMA,
        pltpu.SemaphoreType.DMA,
        pltpu.SemaphoreType.DMA,
        pltpu.SemaphoreType.DMA,
        pltpu.SemaphoreType.DMA,
        pltpu.SemaphoreType.DMA,
        pltpu.SemaphoreType.DMA,
        pltpu.SemaphoreType.DMA,
        pltpu.SemaphoreType.DMA,
        pltpu.SemaphoreType.DMA,
        pltpu.VMEM_SHARED((NPAD, 128), _F32),
    ],
)


def _init_body(x_ref, wn_ref, bn_ref, o0_ref, o1_ref):
    h = _silu(jnp.dot(x_ref[:], wn_ref[:], preferred_element_type=_F32)
              + bn_ref[:])
    o0_ref[:N, :] = h[:, :128]
    o0_ref[N:, :] = jnp.zeros((NPAD - N, 128), _F32)
    o1_ref[:N, :] = h[:, 128:]
    o1_ref[N:, :] = jnp.zeros((NPAD - N, 128), _F32)


_init_call = pl.pallas_call(
    _init_body,
    out_shape=(jax.ShapeDtypeStruct((NPAD, 128), _F32),
               jax.ShapeDtypeStruct((NPAD, 128), _F32)),
)


def _node_body(a0_ref, a1_ref, rdeg_ref, h0_ref, h1_ref, w2_ref, b2_ref,
               wo_ref, bo_ref, g_ref, b_ref, o0_ref, o1_ref):
    agg = jnp.concatenate([a0_ref[:N, :], a1_ref[:N, :]], axis=1) * rdeg_ref[:]
    h = jnp.concatenate([h0_ref[:N, :], h1_ref[:N, :]], axis=1)
    h1_ = _silu(jnp.dot(agg, w2_ref[:], preferred_element_type=_F32)
                + b2_ref[:])
    z = jnp.dot(h1_ + h, wo_ref[:], preferred_element_type=_F32) + bo_ref[:]
    z = jnp.where(z >= 0, z, 0.01 * z)
    mu = jnp.mean(z, axis=0, keepdims=True)
    var = jnp.mean((z - mu) ** 2, axis=0, keepdims=True)
    hn = (z - mu) * lax.rsqrt(var + 1e-5) * g_ref[:] + b_ref[:]
    o0_ref[:N, :] = hn[:, :128]
    o0_ref[N:, :] = jnp.zeros((NPAD - N, 128), _F32)
    o1_ref[:N, :] = hn[:, 128:]
    o1_ref[N:, :] = jnp.zeros((NPAD - N, 128), _F32)


_node_call = pl.pallas_call(
    _node_body,
    out_shape=(jax.ShapeDtypeStruct((NPAD, 128), _F32),
               jax.ShapeDtypeStruct((NPAD, 128), _F32)),
)


def _final_body(h0_ref, h1_ref, wl_ref, bl_ref, wo_ref, bo_ref, batch_ref,
                out_ref):
    h = jnp.concatenate([h0_ref[:N, :], h1_ref[:N, :]], axis=1)
    h = _silu(jnp.dot(h, wl_ref[0], preferred_element_type=_F32)
              + bl_ref[0:1, :])
    h = _silu(jnp.dot(h, wl_ref[1], preferred_element_type=_F32)
              + bl_ref[1:2, :])
    node_out = jnp.dot(h, wo_ref[:], preferred_element_type=_F32) + bo_ref[:]
    onehot = (lax.broadcasted_iota(jnp.int32, (NG, N), 0)
              == batch_ref[:][None, :]).astype(_F32)
    out_ref[:] = jnp.dot(onehot, node_out, preferred_element_type=_F32)


_final_call = pl.pallas_call(
    _final_body,
    out_shape=jax.ShapeDtypeStruct((NG, 1), _F32),
)


def kernel(x, pos, edge_index, edge_attr, batch, W_node, b_node, W_rbf, W_qr,
           W_cat, b_cat, W2, b2, W_o, b_o, bn_g, bn_b, W_lins, b_lins, W_out,
           b_out):
    srcf = jnp.full((EP,), N, jnp.int32).at[:E].set(edge_index[0])
    dstf = jnp.full((EP,), N, jnp.int32).at[:E].set(edge_index[1])
    srcp = srcf.reshape(ER, 128)
    dstp = dstf.reshape(ER, 128)
    src64 = srcf.reshape(EP // 64, 64)
    dst64 = dstf.reshape(EP // 64, 64)
    posx = jnp.zeros((NPAD,), _F32).at[:N].set(pos[:, 0])
    posy = jnp.zeros((NPAD,), _F32).at[:N].set(pos[:, 1])
    posz = jnp.zeros((NPAD,), _F32).at[:N].set(pos[:, 2])

    d2r, degp = _edge_prep(srcp, dstp, posx, posy, posz)
    d2 = d2r.reshape(EP, 1)

    base_s = _base_call(d2, W_rbf, W_qr, W_cat, b_cat.reshape(3, 1, H))

    deg = degp[0] + degp[1]
    rdeg = (1.0 / jnp.maximum(deg[:N], 1.0))[:, None]

    h0, h1 = _init_call(x, W_node, b_node)

    base_flat = base_s.reshape(6 * EP, 128)
    larr = jnp.broadcast_to(jnp.arange(3, dtype=jnp.int32)[:, None],
                            (3, 16))

    def _layer(carry, xs):
        h0, h1 = carry
        lrow, w2, b2l, wo, bo, g, b = xs
        agg = _msg_agg(src64, dst64, h0, h1, base_flat, lrow)
        h0, h1 = _node_call(agg[0], agg[1], rdeg, h0, h1, w2, b2l, wo, bo,
                            g, b)
        return (h0, h1), None

    (h0, h1), _ = lax.scan(_layer, (h0, h1),
                           (larr, W2, b2, W_o, b_o, bn_g, bn_b))

    return _final_call(h0, h1, W_lins, b_lins, W_out, b_out, batch)


# speedup vs baseline: 1.0989x; 1.0989x over previous
import functools

import jax
import jax.numpy as jnp
import numpy as np
from jax import lax
from jax.experimental import pallas as pl
from jax.experimental.pallas import tpu as pltpu
from jax.experimental.pallas import tpu_sc as plsc

N = 10000
E = 320000
H = 256
IN = 48
NG = 64
NPAD = 10240
EP = 327680
ER = EP // 128
BE = 2560

_F32 = jnp.float32


def _silu(v):
    return v * jax.nn.sigmoid(v)


def _base_body(d2_ref, wr_ref, wq_ref, wc_ref, bc_ref, out_ref):
    d2 = d2_ref[:]
    dist = jnp.sqrt(d2 + 1e-12)
    mu = (lax.broadcasted_iota(jnp.int32, (1, 9), 1).astype(_F32)
          * np.float32(0.75))
    sig = np.float32(6.0 / 9)
    rbf = jnp.exp(-(((dist - mu) / sig) ** 2))
    wr = wr_ref[0]
    wq = wq_ref[0]
    wc = wc_ref[0]
    bc = bc_ref[0]
    rbf_h = _silu(jnp.dot(rbf, wr, preferred_element_type=_F32))
    qr_h = _silu(dist * wq)
    bf = jnp.bfloat16
    base = (jnp.dot(rbf_h.astype(bf), wc[:H].astype(bf),
                    preferred_element_type=_F32)
            + jnp.dot(qr_h.astype(bf), wc[H:].astype(bf),
                      preferred_element_type=_F32) + bc)
    out_ref[0, 0] = base[:, :128]
    out_ref[0, 1] = base[:, 128:]


_base_call = pl.pallas_call(
    _base_body,
    grid=(3, EP // BE),
    in_specs=[
        pl.BlockSpec((BE, 1), lambda l, e: (e, 0)),
        pl.BlockSpec((1, 9, H), lambda l, e: (l, 0, 0)),
        pl.BlockSpec((1, 1, H), lambda l, e: (l, 0, 0)),
        pl.BlockSpec((1, 2 * H, H), lambda l, e: (l, 0, 0)),
        pl.BlockSpec((1, 1, H), lambda l, e: (l, 0, 0)),
    ],
    out_specs=pl.BlockSpec((1, 2, BE, 128), lambda l, e: (l, 0, e, 0)),
    out_shape=jax.ShapeDtypeStruct((3, 2, EP, 128), _F32),
)


_SC_MESH = plsc.VectorSubcoreMesh(core_axis_name="c", subcore_axis_name="s",
                                  num_cores=2, num_subcores=16)


def _edge_prep_body(src_ref, dst_ref, px_h, py_h, pz_h, d2_h, degp_h,
                    px, py, pz, srcv, dstv, d2v, onesv, zv, deg_sh):
    c = lax.axis_index("c")
    s = lax.axis_index("s")
    wid = s * 2 + c
    row0 = wid * 80
    pltpu.sync_copy(px_h, px)
    pltpu.sync_copy(py_h, py)
    pltpu.sync_copy(pz_h, pz)
    pltpu.sync_copy(src_ref.at[pl.ds(row0, 80), :], srcv)
    pltpu.sync_copy(dst_ref.at[pl.ds(row0, 80), :], dstv)

    def _zero(i, _):
        zv[pl.ds(i * 16, 16)] = jnp.zeros((16,), _F32)
        return _
    lax.fori_loop(0, 40, _zero, None)
    pltpu.sync_copy(zv, deg_sh.at[pl.ds(s * 640, 640)])

    def _one(i, _):
        onesv[pl.ds(i * 16, 16)] = jnp.full((16,), 1.0, _F32)
        return _
    lax.fori_loop(0, 8, _one, None)
    plsc.subcore_barrier()

    def _blk(k, _):
        def _grp(g, _):
            sl = pl.ds(g * 16, 16)
            si = srcv[k, sl]
            di = dstv[k, sl]
            dx = plsc.load_gather(px, [si]) - plsc.load_gather(px, [di])
            dy = plsc.load_gather(py, [si]) - plsc.load_gather(py, [di])
            dz = plsc.load_gather(pz, [si]) - plsc.load_gather(pz, [di])
            d2v[k, sl] = dx * dx + dy * dy + dz * dz
            return _
        lax.fori_loop(0, 8, _grp, None)
        pltpu.sync_copy(onesv, deg_sh.at[dstv.at[k]], add=True)
        return _
    lax.fori_loop(0, 80, _blk, None)
    pltpu.sync_copy(d2v, d2_h.at[pl.ds(row0, 80), :])
    plsc.subcore_barrier()

    @pl.when(s == 0)
    def _():
        pltpu.sync_copy(deg_sh, degp_h.at[c])


_edge_prep = pl.kernel(
    _edge_prep_body,
    out_type=(jax.ShapeDtypeStruct((ER, 128), _F32),
              jax.ShapeDtypeStruct((2, NPAD), _F32)),
    mesh=_SC_MESH,
    compiler_params=pltpu.CompilerParams(needs_layout_passes=False),
    scratch_types=[
        pltpu.VMEM((NPAD,), _F32),
        pltpu.VMEM((NPAD,), _F32),
        pltpu.VMEM((NPAD,), _F32),
        pltpu.VMEM((80, 128), jnp.int32),
        pltpu.VMEM((80, 128), jnp.int32),
        pltpu.VMEM((80, 128), _F32),
        pltpu.VMEM((128,), _F32),
        pltpu.VMEM((640,), _F32),
        pltpu.VMEM_SHARED((NPAD,), _F32),
    ],
)


_NBLK = 320


def _msg_agg_body(src_ref, dst_ref, h0_ref, h1_ref, base_ref, lrow_ref,
                  out_ref, ibs, ibd, rows, bbuf, lbuf, bidx,
                  is0, is1, is2, is3, gs0, gs1, bs0, bs1, ss0, ss1, agg_sh):
    c = lax.axis_index("c")
    s = lax.axis_index("s")
    blk0 = s * _NBLK
    pltpu.sync_copy(lrow_ref, lbuf)
    iota16 = lax.broadcasted_iota(jnp.int32, (16,), 0)
    off_v = (lbuf[:] * 2 + c) * EP
    isems = (is0, is1, is2, is3)
    gsems = (gs0, gs1)
    bsems = (bs0, bs1)
    ssems = (ss0, ss1)

    def idx_issue(k, j):
        pltpu.async_copy(src_ref.at[blk0 + k], ibs.at[j], isems[j])
        pltpu.async_copy(dst_ref.at[blk0 + k], ibd.at[j], isems[j])

    def idx_wait(j):
        pltpu.make_async_copy(src_ref.at[blk0], ibs.at[j], isems[j]).wait()
        pltpu.make_async_copy(dst_ref.at[blk0], ibd.at[j], isems[j]).wait()

    def gb_issue(k, b, j):
        @pl.when(c == 0)
        def _():
            pltpu.async_copy(h0_ref.at[ibs.at[j]], rows.at[b], gsems[b])

        @pl.when(c == 1)
        def _():
            pltpu.async_copy(h1_ref.at[ibs.at[j]], rows.at[b], gsems[b])
        e0 = (blk0 + k) * 64
        for g in range(4):
            bidx[b, pl.ds(g * 16, 16)] = off_v + (e0 + g * 16) + iota16
        pltpu.async_copy(base_ref.at[bidx.at[b]], bbuf.at[b], bsems[b])

    def gb_wait(b, j):
        pltpu.make_async_copy(h0_ref.at[ibs.at[j]], rows.at[b],
                              gsems[b]).wait()
        pltpu.make_async_copy(base_ref.at[bidx.at[b]], bbuf.at[b],
                              bsems[b]).wait()

    def scat_issue(b, j):
        pltpu.async_copy(rows.at[b], agg_sh.at[ibd.at[j]], ssems[b],
                         add=True)

    def scat_wait(b, j):
        pltpu.make_async_copy(rows.at[b], agg_sh.at[ibd.at[j]],
                              ssems[b]).wait()

    def process(k, r, first=False):
        b = r & 1
        if not first:
            scat_wait(1 - b, (r - 1) & 3)
        if isinstance(k, int):
            if k + 3 < _NBLK:
                idx_issue(k + 3, (r + 3) & 3)
            if k + 1 < _NBLK:
                idx_wait((r + 1) & 3)
                gb_issue(k + 1, 1 - b, (r + 1) & 3)
        else:
            @pl.when(k + 3 < _NBLK)
            def _():
                idx_issue(k + 3, (r + 3) & 3)

            @pl.when(k + 1 < _NBLK)
            def _():
                idx_wait((r + 1) & 3)
                gb_issue(k + 1, 1 - b, (r + 1) & 3)
        gb_wait(b, r)
        @plsc.parallel_loop(0, 64, unroll=2)
        def _mrow(i):
            for j in range(8):
                sl = pl.ds(j * 16, 16)
                rows[b, i, sl] = rows[b, i, sl] * bbuf[b, i, sl]
        scat_issue(b, r)

    @plsc.parallel_loop(0, 64, unroll=2)
    def _zrow(i):
        for j in range(8):
            rows[0, i, pl.ds(j * 16, 16)] = jnp.zeros((16,), _F32)
    for j in range(10):
        pltpu.sync_copy(rows.at[0], agg_sh.at[pl.ds(s * 640 + j * 64, 64), :])
    plsc.subcore_barrier()

    idx_issue(0, 0)
    idx_issue(1, 1)
    idx_issue(2, 2)
    idx_wait(0)
    gb_issue(0, 0, 0)
    process(0, 0, first=True)
    process(1, 1)
    process(2, 2)
    process(3, 3)

    def _sb(it, _):
        k0 = it * 4
        for r in range(4):
            process(k0 + r, r)
        return _
    lax.fori_loop(1, _NBLK // 4, _sb, None)
    scat_wait(1, 3)
    plsc.subcore_barrier()
    pltpu.sync_copy(agg_sh.at[pl.ds(s * 640, 640), :],
                    out_ref.at[c, pl.ds(s * 640, 640), :])


_msg_agg = pl.kernel(
    _msg_agg_body,
    out_type=jax.ShapeDtypeStruct((2, NPAD, 128), _F32),
    mesh=_SC_MESH,
    compiler_params=pltpu.CompilerParams(needs_layout_passes=False),
    scratch_types=[
        pltpu.VMEM((4, 64), jnp.int32),
        pltpu.VMEM((4, 64), jnp.int32),
        pltpu.VMEM((2, 64, 128), _F32),
        pltpu.VMEM((2, 64, 128), _F32),
        pltpu.VMEM((16,), jnp.int32),
        pltpu.VMEM((2, 64), jnp.int32),
        pltpu.SemaphoreType.DMA,
        pltpu.SemaphoreType.DMA,
        pltpu.SemaphoreType.DMA,
        pltpu.SemaphoreType.DMA,
        pltpu.SemaphoreType.DMA,
        pltpu.SemaphoreType.DMA,
        pltpu.SemaphoreType.DMA,
        pltpu.SemaphoreType.DMA,
        pltpu.SemaphoreType.DMA,
        pltpu.SemaphoreType.DMA,
        pltpu.VMEM_SHARED((NPAD, 128), _F32),
    ],
)


def _init_body(x_ref, wn_ref, bn_ref, o0_ref, o1_ref):
    h = _silu(jnp.dot(x_ref[:], wn_ref[:], preferred_element_type=_F32)
              + bn_ref[:])
    o0_ref[:N, :] = h[:, :128]
    o0_ref[N:, :] = jnp.zeros((NPAD - N, 128), _F32)
    o1_ref[:N, :] = h[:, 128:]
    o1_ref[N:, :] = jnp.zeros((NPAD - N, 128), _F32)


_init_call = pl.pallas_call(
    _init_body,
    out_shape=(jax.ShapeDtypeStruct((NPAD, 128), _F32),
               jax.ShapeDtypeStruct((NPAD, 128), _F32)),
)


def _node_body(a0_ref, a1_ref, rdeg_ref, h0_ref, h1_ref, w2_ref, b2_ref,
               wo_ref, bo_ref, g_ref, b_ref, o0_ref, o1_ref):
    agg = jnp.concatenate([a0_ref[:N, :], a1_ref[:N, :]], axis=1) * rdeg_ref[:]
    h = jnp.concatenate([h0_ref[:N, :], h1_ref[:N, :]], axis=1)
    h1_ = _silu(jnp.dot(agg, w2_ref[:], preferred_element_type=_F32)
                + b2_ref[:])
    z = jnp.dot(h1_ + h, wo_ref[:], preferred_element_type=_F32) + bo_ref[:]
    z = jnp.where(z >= 0, z, 0.01 * z)
    mu = jnp.mean(z, axis=0, keepdims=True)
    var = jnp.mean((z - mu) ** 2, axis=0, keepdims=True)
    hn = (z - mu) * lax.rsqrt(var + 1e-5) * g_ref[:] + b_ref[:]
    o0_ref[:N, :] = hn[:, :128]
    o0_ref[N:, :] = jnp.zeros((NPAD - N, 128), _F32)
    o1_ref[:N, :] = hn[:, 128:]
    o1_ref[N:, :] = jnp.zeros((NPAD - N, 128), _F32)


_node_call = pl.pallas_call(
    _node_body,
    out_shape=(jax.ShapeDtypeStruct((NPAD, 128), _F32),
               jax.ShapeDtypeStruct((NPAD, 128), _F32)),
)


def _final_body(h0_ref, h1_ref, wl_ref, bl_ref, wo_ref, bo_ref, batch_ref,
                out_ref):
    h = jnp.concatenate([h0_ref[:N, :], h1_ref[:N, :]], axis=1)
    h = _silu(jnp.dot(h, wl_ref[0], preferred_element_type=_F32)
              + bl_ref[0:1, :])
    h = _silu(jnp.dot(h, wl_ref[1], preferred_element_type=_F32)
              + bl_ref[1:2, :])
    node_out = jnp.dot(h, wo_ref[:], preferred_element_type=_F32) + bo_ref[:]
    onehot = (lax.broadcasted_iota(jnp.int32, (NG, N), 0)
              == batch_ref[:][None, :]).astype(_F32)
    out_ref[:] = jnp.dot(onehot, node_out, preferred_element_type=_F32)


_final_call = pl.pallas_call(
    _final_body,
    out_shape=jax.ShapeDtypeStruct((NG, 1), _F32),
)


def kernel(x, pos, edge_index, edge_attr, batch, W_node, b_node, W_rbf, W_qr,
           W_cat, b_cat, W2, b2, W_o, b_o, bn_g, bn_b, W_lins, b_lins, W_out,
           b_out):
    srcf = jnp.full((EP,), N, jnp.int32).at[:E].set(edge_index[0])
    dstf = jnp.full((EP,), N, jnp.int32).at[:E].set(edge_index[1])
    srcp = srcf.reshape(ER, 128)
    dstp = dstf.reshape(ER, 128)
    src64 = srcf.reshape(EP // 64, 64)
    dst64 = dstf.reshape(EP // 64, 64)
    posx = jnp.zeros((NPAD,), _F32).at[:N].set(pos[:, 0])
    posy = jnp.zeros((NPAD,), _F32).at[:N].set(pos[:, 1])
    posz = jnp.zeros((NPAD,), _F32).at[:N].set(pos[:, 2])

    d2r, degp = _edge_prep(srcp, dstp, posx, posy, posz)
    d2 = d2r.reshape(EP, 1)

    base_s = _base_call(d2, W_rbf, W_qr, W_cat, b_cat.reshape(3, 1, H))

    deg = degp[0] + degp[1]
    rdeg = (1.0 / jnp.maximum(deg[:N], 1.0))[:, None]

    h0, h1 = _init_call(x, W_node, b_node)

    base_flat = base_s.reshape(6 * EP, 128)
    larr = jnp.broadcast_to(jnp.arange(3, dtype=jnp.int32)[:, None],
                            (3, 16))

    def _layer(carry, xs):
        h0, h1 = carry
        lrow, w2, b2l, wo, bo, g, b = xs
        agg = _msg_agg(src64, dst64, h0, h1, base_flat, lrow)
        h0, h1 = _node_call(agg[0], agg[1], rdeg, h0, h1, w2, b2l, wo, bo,
                            g, b)
        return (h0, h1), None

    (h0, h1), _ = lax.scan(_layer, (h0, h1),
                           (larr, W2, b2, W_o, b_o, bn_g, bn_b))

    return _final_call(h0, h1, W_lins, b_lins, W_out, b_out, batch)
